# baseline (speedup 1.0000x reference)
import functools

import jax
import jax.numpy as jnp
from jax import lax
from jax.experimental import pallas as pl
from jax.experimental.pallas import tpu as pltpu
from jax.experimental.pallas import tpu_sc as plsc

N = 10000
E = 320000
D = 128
NC = 2
NS = 16
NW = NC * NS
EPT = E // NW
CH = 80
NCHUNK = EPT // CH
CH_E = 128
PEPT = 10112
NPAD = PEPT - EPT
NCHUNK_E = PEPT // CH_E
NA = N + 8
R = 1000
NB = N // R


@functools.partial(
    pl.kernel,
    out_type=jax.ShapeDtypeStruct((NC, N), jnp.float32),
    mesh=plsc.VectorSubcoreMesh(core_axis_name="c", subcore_axis_name="s"),
    scratch_types=[
        pltpu.VMEM((NCHUNK, CH), jnp.int32),
        pltpu.VMEM((CH,), jnp.float32),
        pltpu.VMEM_SHARED((N,), jnp.float32),
    ],
)
def _deg_kernel(dst_hbm, zeros_hbm, hist_hbm, dstbuf, ones, deg_sh):
    c = lax.axis_index("c")
    s = lax.axis_index("s")
    w = c * NS + s
    pltpu.sync_copy(dst_hbm.at[w], dstbuf)
    for i in range(CH // 16):
        ones[pl.ds(i * 16, 16)] = jnp.ones((16,), jnp.float32)

    @pl.when(s == 0)
    def _():
        pltpu.sync_copy(zeros_hbm, deg_sh)

    plsc.subcore_barrier()

    def chunk(j, carry):
        pltpu.sync_copy(ones, deg_sh.at[dstbuf.at[j]], add=True)
        return carry

    lax.fori_loop(0, NCHUNK, chunk, 0)
    plsc.subcore_barrier()

    @pl.when(s == 0)
    def _():
        pltpu.sync_copy(deg_sh, hist_hbm.at[c])


@functools.partial(
    pl.kernel,
    out_type=jax.ShapeDtypeStruct((NC, N, D), jnp.float32),
    mesh=plsc.VectorSubcoreMesh(core_axis_name="c", subcore_axis_name="s"),
    scratch_types=[
        pltpu.VMEM((CH_E,), jnp.int32),
        pltpu.VMEM((CH_E,), jnp.int32),
        pltpu.VMEM((CH_E,), jnp.int32),
        pltpu.VMEM((CH_E,), jnp.int32),
        pltpu.VMEM((CH_E,), jnp.int32),
        pltpu.VMEM((CH_E,), jnp.int32),
        pltpu.VMEM((CH_E, D), jnp.float32),
        pltpu.VMEM((CH_E, D), jnp.float32),
        pltpu.VMEM((CH_E, D), jnp.float32),
        pltpu.VMEM_SHARED((NA, D), jnp.float32),
        pltpu.SemaphoreType.DMA,
        pltpu.SemaphoreType.DMA,
        pltpu.SemaphoreType.DMA,
        pltpu.SemaphoreType.DMA,
        pltpu.SemaphoreType.DMA,
        pltpu.SemaphoreType.DMA,
    ],
)
def _edge_kernel(y_hbm, src_hbm, dst_hbm, zeros_hbm, acc_hbm,
                 sidx0, sidx1, sidx2, didx0, didx1, didx2,
                 rows0, rows1, rows2, acc_sh,
                 semi0, semi1, semi2, semg0, semg1, semg2):
    c = lax.axis_index("c")
    s = lax.axis_index("s")
    w = c * NS + s
    sidx = (sidx0, sidx1, sidx2)
    didx = (didx0, didx1, didx2)
    rows = (rows0, rows1, rows2)
    semi = (semi0, semi1, semi2)
    semg = (semg0, semg1, semg2)

    def idx_issue(j, q):
        pltpu.async_copy(src_hbm.at[w].at[j], sidx[q], semi[q])
        pltpu.async_copy(dst_hbm.at[w].at[j], didx[q], semi[q])

    def idx_wait(j, q):
        pltpu.make_async_copy(src_hbm.at[w].at[j], sidx[q], semi[q]).wait()
        pltpu.make_async_copy(dst_hbm.at[w].at[j], didx[q], semi[q]).wait()

    for q in range(3):
        idx_issue(q, q)

    @pl.when(s < NS - 1)
    def _():
        pltpu.sync_copy(zeros_hbm.at[pl.ds(s * 624, 624)],
                        acc_sh.at[pl.ds(s * 624, 624)])

    @pl.when(s == NS - 1)
    def _():
        pltpu.sync_copy(zeros_hbm.at[pl.ds(9360, 640)],
                        acc_sh.at[pl.ds(9360, 640)])

    plsc.subcore_barrier()

    for q in range(2):
        idx_wait(q, q)
        pltpu.async_copy(y_hbm.at[sidx[q]], rows[q], semg[q])

    def chunk_triple(p, carry):
        j3 = p * 3
        for q in range(3):
            j = j3 + q
            pltpu.make_async_copy(y_hbm.at[sidx[q]], rows[q],
                                  semg[q]).wait()
            pltpu.sync_copy(rows[q], acc_sh.at[didx[q]], add=True)

            @pl.when(j + 3 < NCHUNK_E)
            def _():
                idx_issue(j + 3, q)

            q2 = (q + 2) % 3

            @pl.when(j + 2 < NCHUNK_E)
            def _():
                idx_wait(j + 2, q2)
                pltpu.async_copy(y_hbm.at[sidx[q2]], rows[q2], semg[q2])

        return carry

    lax.fori_loop(0, NCHUNK_E // 3, chunk_triple, 0)
    for q in range(NCHUNK_E % 3):
        pltpu.make_async_copy(y_hbm.at[sidx[q]], rows[q], semg[q]).wait()
        pltpu.sync_copy(rows[q], acc_sh.at[didx[q]], add=True)
    plsc.subcore_barrier()

    @pl.when(s < NS - 1)
    def _():
        pltpu.sync_copy(acc_sh.at[pl.ds(s * 624, 624)],
                        acc_hbm.at[c].at[pl.ds(s * 624, 624)])

    @pl.when(s == NS - 1)
    def _():
        pltpu.sync_copy(acc_sh.at[pl.ds(9360, 640)],
                        acc_hbm.at[c].at[pl.ds(9360, 640)])


def _dinv_body(hist_ref, dinv_ref):
    deg = jnp.sum(hist_ref[...], axis=0, keepdims=True)
    dinv_ref[...] = lax.rsqrt(deg + 1.0)


def _tc_dinv(hist):
    return pl.pallas_call(
        _dinv_body,
        in_specs=[pl.BlockSpec((NC, N), lambda: (0, 0))],
        out_specs=pl.BlockSpec((1, N), lambda: (0, 0)),
        out_shape=jax.ShapeDtypeStruct((1, N), jnp.float32),
    )(hist)


def _k1_body(x_ref, w_ref, dinv_ref, y_ref):
    y_ref[...] = dinv_ref[...] * jnp.dot(x_ref[...], w_ref[...],
                                         preferred_element_type=jnp.float32)


def _tc_layer1(x, W1, dinv_col):
    return pl.pallas_call(
        _k1_body,
        grid=(NB,),
        in_specs=[
            pl.BlockSpec((R, D), lambda i: (i, 0)),
            pl.BlockSpec((D, D), lambda i: (0, 0)),
            pl.BlockSpec((R, 1), lambda i: (i, 0)),
        ],
        out_specs=pl.BlockSpec((R, D), lambda i: (i, 0)),
        out_shape=jax.ShapeDtypeStruct((N, D), jnp.float32),
    )(x, W1, dinv_col)


def _k2_body(acca_ref, accb_ref, y_ref, dinv_ref, b1_ref, w_ref, y2_ref):
    dinv = dinv_ref[...]
    h = dinv * (acca_ref[...] + accb_ref[...] + y_ref[...]) + b1_ref[...]
    h = jnp.maximum(h, 0.0)
    y2_ref[...] = dinv * jnp.dot(h, w_ref[...],
                                 preferred_element_type=jnp.float32)


def _tc_layer2(acc1, y1, dinv_col, b1, W2):
    return pl.pallas_call(
        _k2_body,
        grid=(NB,),
        in_specs=[
            pl.BlockSpec((R, D), lambda i: (i, 0)),
            pl.BlockSpec((R, D), lambda i: (NB + i, 0)),
            pl.BlockSpec((R, D), lambda i: (i, 0)),
            pl.BlockSpec((R, 1), lambda i: (i, 0)),
            pl.BlockSpec((1, D), lambda i: (0, 0)),
            pl.BlockSpec((D, D), lambda i: (0, 0)),
        ],
        out_specs=pl.BlockSpec((R, D), lambda i: (i, 0)),
        out_shape=jax.ShapeDtypeStruct((N, D), jnp.float32),
    )(acc1, acc1, y1, dinv_col, b1, W2)


def _k3_body(acca_ref, accb_ref, y_ref, dinv_ref, b2_ref, out_ref):
    out_ref[...] = (dinv_ref[...] *
                    (acca_ref[...] + accb_ref[...] + y_ref[...]) + b2_ref[...])


def _tc_final(acc2, y2, dinv_col, b2):
    return pl.pallas_call(
        _k3_body,
        grid=(NB,),
        in_specs=[
            pl.BlockSpec((R, D), lambda i: (i, 0)),
            pl.BlockSpec((R, D), lambda i: (NB + i, 0)),
            pl.BlockSpec((R, D), lambda i: (i, 0)),
            pl.BlockSpec((R, 1), lambda i: (i, 0)),
            pl.BlockSpec((1, D), lambda i: (0, 0)),
        ],
        out_specs=pl.BlockSpec((R, D), lambda i: (i, 0)),
        out_shape=jax.ShapeDtypeStruct((N, D), jnp.float32),
    )(acc2, acc2, y2, dinv_col, b2)


def kernel(x, edge_index, W1, b1, W2, b2):
    src = edge_index[0].astype(jnp.int32)
    dst = edge_index[1].astype(jnp.int32)
    pad_src = jnp.zeros((NW, NPAD), jnp.int32)
    pad_dst = jnp.full((NW, NPAD), N, jnp.int32)
    src3 = jnp.concatenate([src.reshape(NW, EPT), pad_src],
                           axis=1).reshape(NW, NCHUNK_E, CH_E)
    dst2 = jnp.concatenate([dst.reshape(NW, EPT), pad_dst],
                           axis=1).reshape(NW, NCHUNK_E, CH_E)
    dst3 = dst.reshape(NW, NCHUNK, CH)
    zeros1 = jnp.zeros((N,), jnp.float32)
    zeros2 = jnp.zeros((N, D), jnp.float32)
    b1r = b1.reshape(1, D)
    b2r = b2.reshape(1, D)

    hist = _deg_kernel(dst3, zeros1)
    dinv_col = _tc_dinv(hist).reshape(N, 1)

    y1 = _tc_layer1(x, W1, dinv_col)
    acc1 = _edge_kernel(y1, src3, dst2, zeros2).reshape(NC * N, D)
    y2 = _tc_layer2(acc1, y1, dinv_col, b1r, W2)
    acc2 = _edge_kernel(y2, src3, dst2, zeros2).reshape(NC * N, D)
    return _tc_final(acc2, y2, dinv_col, b2r)


# speedup vs baseline: 1.9263x; 1.9263x over previous
import functools

import jax
import jax.numpy as jnp
from jax import lax
from jax.experimental import pallas as pl
from jax.experimental.pallas import tpu as pltpu
from jax.experimental.pallas import tpu_sc as plsc

N = 10000
E = 320000
D = 128
NC = 2
NS = 16
NW = NC * NS
EPT = E // NW
CH = 80
NCHUNK = EPT // CH
R = 1000
NB = N // R


@functools.partial(
    pl.kernel,
    out_type=jax.ShapeDtypeStruct((NC, N), jnp.float32),
    mesh=plsc.VectorSubcoreMesh(core_axis_name="c", subcore_axis_name="s"),
    scratch_types=[
        pltpu.VMEM((NCHUNK, CH), jnp.int32),
        pltpu.VMEM((CH,), jnp.float32),
        pltpu.VMEM_SHARED((N,), jnp.float32),
    ],
)
def _deg_kernel(dst_hbm, zeros_hbm, hist_hbm, dstbuf, ones, deg_sh):
    c = lax.axis_index("c")
    s = lax.axis_index("s")
    w = c * NS + s
    pltpu.sync_copy(dst_hbm.at[w], dstbuf)
    for i in range(CH // 16):
        ones[pl.ds(i * 16, 16)] = jnp.ones((16,), jnp.float32)

    @pl.when(s == 0)
    def _():
        pltpu.sync_copy(zeros_hbm, deg_sh)

    plsc.subcore_barrier()

    def chunk(j, carry):
        pltpu.sync_copy(ones, deg_sh.at[dstbuf.at[j]], add=True)
        return carry

    lax.fori_loop(0, NCHUNK, chunk, 0)
    plsc.subcore_barrier()

    @pl.when(s == 0)
    def _():
        pltpu.sync_copy(deg_sh, hist_hbm.at[c])


@functools.partial(
    pl.kernel,
    out_type=jax.ShapeDtypeStruct((NC, N, D), jnp.float32),
    mesh=plsc.VectorSubcoreMesh(core_axis_name="c", subcore_axis_name="s"),
    scratch_types=[
        pltpu.VMEM((EPT,), jnp.int32),
        pltpu.VMEM((EPT,), jnp.int32),
        pltpu.VMEM((CH, D), jnp.float32),
        pltpu.VMEM((CH, D), jnp.float32),
        pltpu.VMEM((CH, D), jnp.float32),
        pltpu.VMEM_SHARED((N, D), jnp.float32),
        pltpu.SemaphoreType.DMA,
        pltpu.SemaphoreType.DMA,
        pltpu.SemaphoreType.DMA,
        pltpu.SemaphoreType.DMA,
        pltpu.SemaphoreType.DMA,
        pltpu.SemaphoreType.DMA,
    ],
)
def _edge_kernel(y_hbm, src_hbm, dst_hbm, zeros_hbm, acc_hbm,
                 srcbuf, dstbuf, rows0, rows1, rows2, acc_sh,
                 semg0, semg1, semg2, sems0, sems1, sems2):
    c = lax.axis_index("c")
    s = lax.axis_index("s")
    w = c * NS + s
    pltpu.sync_copy(src_hbm.at[w], srcbuf)
    pltpu.sync_copy(dst_hbm.at[w], dstbuf)

    @pl.when(s < NS - 1)
    def _():
        pltpu.sync_copy(zeros_hbm.at[pl.ds(s * 624, 624)],
                        acc_sh.at[pl.ds(s * 624, 624)])

    @pl.when(s == NS - 1)
    def _():
        pltpu.sync_copy(zeros_hbm.at[pl.ds(9360, 640)],
                        acc_sh.at[pl.ds(9360, 640)])

    plsc.subcore_barrier()

    def sidx(j):
        return srcbuf.at[pl.ds(j * CH, CH)]

    def didx(j):
        return dstbuf.at[pl.ds(j * CH, CH)]

    bufs = ((rows0, semg0, sems0), (rows1, semg1, sems1),
            (rows2, semg2, sems2))

    def gwait_scatter(j, q):
        rows, semg, sems = bufs[q]
        pltpu.make_async_copy(y_hbm.at[sidx(j)], rows, semg).wait()
        pltpu.async_copy(rows, acc_sh.at[didx(j)], sems, add=True)

    def swait_gissue(j, q):
        rows, semg, sems = bufs[q]
        pltpu.make_async_copy(rows, acc_sh.at[didx(j - 2)], sems).wait()
        pltpu.async_copy(y_hbm.at[sidx(j)], rows, semg)

    for q in range(3):
        pltpu.async_copy(y_hbm.at[sidx(q)], bufs[q][0], bufs[q][1])
    gwait_scatter(0, 0)
    gwait_scatter(1, 1)
    swait_gissue(3, 0)
    gwait_scatter(2, 2)
    swait_gissue(4, 1)

    def chunk_triple(p, carry):
        j3 = p * 3
        for q in range(3):
            j = j3 + q
            gwait_scatter(j, q)

            @pl.when(j + 2 < NCHUNK)
            def _():
                swait_gissue(j + 2, (q + 2) % 3)

        return carry

    lax.fori_loop(1, NCHUNK // 3, chunk_triple, 0)
    for q in range(NCHUNK % 3):
        j = (NCHUNK // 3) * 3 + q
        gwait_scatter(j, j % 3)
    for j in range(NCHUNK - 3, NCHUNK):
        rows, semg, sems = bufs[j % 3]
        pltpu.make_async_copy(rows, acc_sh.at[didx(j)], sems).wait()
    plsc.subcore_barrier()

    @pl.when(s < NS - 1)
    def _():
        pltpu.sync_copy(acc_sh.at[pl.ds(s * 624, 624)],
                        acc_hbm.at[c].at[pl.ds(s * 624, 624)])

    @pl.when(s == NS - 1)
    def _():
        pltpu.sync_copy(acc_sh.at[pl.ds(9360, 640)],
                        acc_hbm.at[c].at[pl.ds(9360, 640)])


def _dinv_body(hist_ref, dinv_ref):
    deg = jnp.sum(hist_ref[...], axis=0, keepdims=True)
    dinv_ref[...] = lax.rsqrt(deg + 1.0)


def _tc_dinv(hist):
    return pl.pallas_call(
        _dinv_body,
        in_specs=[pl.BlockSpec((NC, N), lambda: (0, 0))],
        out_specs=pl.BlockSpec((1, N), lambda: (0, 0)),
        out_shape=jax.ShapeDtypeStruct((1, N), jnp.float32),
    )(hist)


def _k1_body(x_ref, w_ref, dinv_ref, y_ref):
    y_ref[...] = dinv_ref[...] * jnp.dot(x_ref[...], w_ref[...],
                                         preferred_element_type=jnp.float32)


def _tc_layer1(x, W1, dinv_col):
    return pl.pallas_call(
        _k1_body,
        grid=(NB,),
        in_specs=[
            pl.BlockSpec((R, D), lambda i: (i, 0)),
            pl.BlockSpec((D, D), lambda i: (0, 0)),
            pl.BlockSpec((R, 1), lambda i: (i, 0)),
        ],
        out_specs=pl.BlockSpec((R, D), lambda i: (i, 0)),
        out_shape=jax.ShapeDtypeStruct((N, D), jnp.float32),
    )(x, W1, dinv_col)


def _k2_body(acca_ref, accb_ref, y_ref, dinv_ref, b1_ref, w_ref, y2_ref):
    dinv = dinv_ref[...]
    h = dinv * (acca_ref[...] + accb_ref[...] + y_ref[...]) + b1_ref[...]
    h = jnp.maximum(h, 0.0)
    y2_ref[...] = dinv * jnp.dot(h, w_ref[...],
                                 preferred_element_type=jnp.float32)


def _tc_layer2(acc1, y1, dinv_col, b1, W2):
    return pl.pallas_call(
        _k2_body,
        grid=(NB,),
        in_specs=[
            pl.BlockSpec((R, D), lambda i: (i, 0)),
            pl.BlockSpec((R, D), lambda i: (NB + i, 0)),
            pl.BlockSpec((R, D), lambda i: (i, 0)),
            pl.BlockSpec((R, 1), lambda i: (i, 0)),
            pl.BlockSpec((1, D), lambda i: (0, 0)),
            pl.BlockSpec((D, D), lambda i: (0, 0)),
        ],
        out_specs=pl.BlockSpec((R, D), lambda i: (i, 0)),
        out_shape=jax.ShapeDtypeStruct((N, D), jnp.float32),
    )(acc1, acc1, y1, dinv_col, b1, W2)


def _k3_body(acca_ref, accb_ref, y_ref, dinv_ref, b2_ref, out_ref):
    out_ref[...] = (dinv_ref[...] *
                    (acca_ref[...] + accb_ref[...] + y_ref[...]) + b2_ref[...])


def _tc_final(acc2, y2, dinv_col, b2):
    return pl.pallas_call(
        _k3_body,
        grid=(NB,),
        in_specs=[
            pl.BlockSpec((R, D), lambda i: (i, 0)),
            pl.BlockSpec((R, D), lambda i: (NB + i, 0)),
            pl.BlockSpec((R, D), lambda i: (i, 0)),
            pl.BlockSpec((R, 1), lambda i: (i, 0)),
            pl.BlockSpec((1, D), lambda i: (0, 0)),
        ],
        out_specs=pl.BlockSpec((R, D), lambda i: (i, 0)),
        out_shape=jax.ShapeDtypeStruct((N, D), jnp.float32),
    )(acc2, acc2, y2, dinv_col, b2)


def kernel(x, edge_index, W1, b1, W2, b2):
    src = edge_index[0].astype(jnp.int32)
    dst = edge_index[1].astype(jnp.int32)
    src3 = src.reshape(NW, EPT)
    dst2 = dst.reshape(NW, EPT)
    dst3 = dst.reshape(NW, NCHUNK, CH)
    zeros1 = jnp.zeros((N,), jnp.float32)
    zeros2 = jnp.zeros((N, D), jnp.float32)
    b1r = b1.reshape(1, D)
    b2r = b2.reshape(1, D)

    hist = _deg_kernel(dst3, zeros1)
    dinv_col = _tc_dinv(hist).reshape(N, 1)

    y1 = _tc_layer1(x, W1, dinv_col)
    acc1 = _edge_kernel(y1, src3, dst2, zeros2).reshape(NC * N, D)
    y2 = _tc_layer2(acc1, y1, dinv_col, b1r, W2)
    acc2 = _edge_kernel(y2, src3, dst2, zeros2).reshape(NC * N, D)
    return _tc_final(acc2, y2, dinv_col, b2r)


# speedup vs baseline: 1.9574x; 1.0161x over previous
import functools

import jax
import jax.numpy as jnp
from jax import lax
from jax.experimental import pallas as pl
from jax.experimental.pallas import tpu as pltpu
from jax.experimental.pallas import tpu_sc as plsc

N = 10000
E = 320000
D = 128
NC = 2
NS = 16
NW = NC * NS
EPT = E // NW
CH = 80
NCHUNK = EPT // CH
R = 1000
NB = N // R


@functools.partial(
    pl.kernel,
    out_type=jax.ShapeDtypeStruct((NC, N), jnp.float32),
    mesh=plsc.VectorSubcoreMesh(core_axis_name="c", subcore_axis_name="s"),
    scratch_types=[
        pltpu.VMEM((NCHUNK, CH), jnp.int32),
        pltpu.VMEM((CH,), jnp.float32),
        pltpu.VMEM_SHARED((N,), jnp.float32),
    ],
)
def _deg_kernel(dst_hbm, zeros_hbm, hist_hbm, dstbuf, ones, deg_sh):
    c = lax.axis_index("c")
    s = lax.axis_index("s")
    w = c * NS + s
    pltpu.sync_copy(dst_hbm.at[w], dstbuf)
    for i in range(CH // 16):
        ones[pl.ds(i * 16, 16)] = jnp.ones((16,), jnp.float32)

    @pl.when(s == 0)
    def _():
        pltpu.sync_copy(zeros_hbm, deg_sh)

    plsc.subcore_barrier()

    def chunk(j, carry):
        pltpu.sync_copy(ones, deg_sh.at[dstbuf.at[j]], add=True)
        return carry

    lax.fori_loop(0, NCHUNK, chunk, 0)
    plsc.subcore_barrier()

    @pl.when(s == 0)
    def _():
        pltpu.sync_copy(deg_sh, hist_hbm.at[c])


@functools.partial(
    pl.kernel,
    out_type=jax.ShapeDtypeStruct((NC, N, D), jnp.float32),
    mesh=plsc.VectorSubcoreMesh(core_axis_name="c", subcore_axis_name="s"),
    scratch_types=[
        pltpu.VMEM((EPT,), jnp.int32),
        pltpu.VMEM((EPT,), jnp.int32),
        pltpu.VMEM((CH, D), jnp.float32),
        pltpu.VMEM((CH, D), jnp.float32),
        pltpu.VMEM((CH, D), jnp.float32),
        pltpu.VMEM_SHARED((N, D), jnp.float32),
        pltpu.SemaphoreType.DMA,
        pltpu.SemaphoreType.DMA,
        pltpu.SemaphoreType.DMA,
    ],
)
def _edge_kernel(y_hbm, src_hbm, dst_hbm, zeros_hbm, acc_hbm,
                 srcbuf, dstbuf, rows0, rows1, rows2, acc_sh,
                 semg0, semg1, semg2):
    c = lax.axis_index("c")
    s = lax.axis_index("s")
    w = c * NS + s
    pltpu.sync_copy(src_hbm.at[w], srcbuf)
    pltpu.sync_copy(dst_hbm.at[w], dstbuf)

    @pl.when(s < NS - 1)
    def _():
        pltpu.sync_copy(zeros_hbm.at[pl.ds(s * 624, 624)],
                        acc_sh.at[pl.ds(s * 624, 624)])

    @pl.when(s == NS - 1)
    def _():
        pltpu.sync_copy(zeros_hbm.at[pl.ds(9360, 640)],
                        acc_sh.at[pl.ds(9360, 640)])

    plsc.subcore_barrier()

    def sidx(j):
        return srcbuf.at[pl.ds(j * CH, CH)]

    def didx(j):
        return dstbuf.at[pl.ds(j * CH, CH)]

    bufs = ((rows0, semg0), (rows1, semg1), (rows2, semg2))

    for q in range(3):
        pltpu.async_copy(y_hbm.at[sidx(q)], bufs[q][0], bufs[q][1])

    def chunk_triple(p, carry):
        j3 = p * 3
        for q in range(3):
            j = j3 + q
            rows, sem = bufs[q]
            pltpu.make_async_copy(y_hbm.at[sidx(j)], rows, sem).wait()
            pltpu.sync_copy(rows, acc_sh.at[didx(j)], add=True)

            @pl.when(j < NCHUNK - 3)
            def _():
                pltpu.async_copy(y_hbm.at[sidx(j + 3)], rows, sem)

        return carry

    lax.fori_loop(0, NCHUNK // 3, chunk_triple, 0)
    for q in range(NCHUNK % 3):
        j = (NCHUNK // 3) * 3 + q
        rows, sem = bufs[q]
        pltpu.make_async_copy(y_hbm.at[sidx(j)], rows, sem).wait()
        pltpu.sync_copy(rows, acc_sh.at[didx(j)], add=True)
    plsc.subcore_barrier()

    @pl.when(s < NS - 1)
    def _():
        pltpu.sync_copy(acc_sh.at[pl.ds(s * 624, 624)],
                        acc_hbm.at[c].at[pl.ds(s * 624, 624)])

    @pl.when(s == NS - 1)
    def _():
        pltpu.sync_copy(acc_sh.at[pl.ds(9360, 640)],
                        acc_hbm.at[c].at[pl.ds(9360, 640)])


_HIST_SPEC = pl.BlockSpec((NC, R, 1), lambda i: (0, i, 0))


def _dinv_of(hist_ref):
    return lax.rsqrt(hist_ref[0] + hist_ref[1] + 1.0)


def _k1_body(x_ref, w_ref, hist_ref, y_ref):
    y_ref[...] = _dinv_of(hist_ref) * jnp.dot(
        x_ref[...], w_ref[...], preferred_element_type=jnp.float32)


def _tc_layer1(x, W1, hist3):
    return pl.pallas_call(
        _k1_body,
        grid=(NB,),
        in_specs=[
            pl.BlockSpec((R, D), lambda i: (i, 0)),
            pl.BlockSpec((D, D), lambda i: (0, 0)),
            _HIST_SPEC,
        ],
        out_specs=pl.BlockSpec((R, D), lambda i: (i, 0)),
        out_shape=jax.ShapeDtypeStruct((N, D), jnp.float32),
    )(x, W1, hist3)


def _k2_body(acca_ref, accb_ref, y_ref, hist_ref, b1_ref, w_ref, y2_ref):
    dinv = _dinv_of(hist_ref)
    h = dinv * (acca_ref[...] + accb_ref[...] + y_ref[...]) + b1_ref[...]
    h = jnp.maximum(h, 0.0)
    y2_ref[...] = dinv * jnp.dot(h, w_ref[...],
                                 preferred_element_type=jnp.float32)


def _tc_layer2(acc1, y1, hist3, b1, W2):
    return pl.pallas_call(
        _k2_body,
        grid=(NB,),
        in_specs=[
            pl.BlockSpec((R, D), lambda i: (i, 0)),
            pl.BlockSpec((R, D), lambda i: (NB + i, 0)),
            pl.BlockSpec((R, D), lambda i: (i, 0)),
            _HIST_SPEC,
            pl.BlockSpec((1, D), lambda i: (0, 0)),
            pl.BlockSpec((D, D), lambda i: (0, 0)),
        ],
        out_specs=pl.BlockSpec((R, D), lambda i: (i, 0)),
        out_shape=jax.ShapeDtypeStruct((N, D), jnp.float32),
    )(acc1, acc1, y1, hist3, b1, W2)


def _k3_body(acca_ref, accb_ref, y_ref, hist_ref, b2_ref, out_ref):
    out_ref[...] = (_dinv_of(hist_ref) *
                    (acca_ref[...] + accb_ref[...] + y_ref[...]) + b2_ref[...])


def _tc_final(acc2, y2, hist3, b2):
    return pl.pallas_call(
        _k3_body,
        grid=(NB,),
        in_specs=[
            pl.BlockSpec((R, D), lambda i: (i, 0)),
            pl.BlockSpec((R, D), lambda i: (NB + i, 0)),
            pl.BlockSpec((R, D), lambda i: (i, 0)),
            _HIST_SPEC,
            pl.BlockSpec((1, D), lambda i: (0, 0)),
        ],
        out_specs=pl.BlockSpec((R, D), lambda i: (i, 0)),
        out_shape=jax.ShapeDtypeStruct((N, D), jnp.float32),
    )(acc2, acc2, y2, hist3, b2)


def kernel(x, edge_index, W1, b1, W2, b2):
    src = edge_index[0].astype(jnp.int32)
    dst = edge_index[1].astype(jnp.int32)
    src3 = src.reshape(NW, EPT)
    dst2 = dst.reshape(NW, EPT)
    dst3 = dst.reshape(NW, NCHUNK, CH)
    zeros1 = jnp.zeros((N,), jnp.float32)
    zeros2 = jnp.zeros((N, D), jnp.float32)
    b1r = b1.reshape(1, D)
    b2r = b2.reshape(1, D)

    hist3 = _deg_kernel(dst3, zeros1).reshape(NC, N, 1)

    y1 = _tc_layer1(x, W1, hist3)
    acc1 = _edge_kernel(y1, src3, dst2, zeros2).reshape(NC * N, D)
    y2 = _tc_layer2(acc1, y1, hist3, b1r, W2)
    acc2 = _edge_kernel(y2, src3, dst2, zeros2).reshape(NC * N, D)
    return _tc_final(acc2, y2, hist3, b2r)


# speedup vs baseline: 2.0238x; 1.0339x over previous
import functools

import jax
import jax.numpy as jnp
from jax import lax
from jax.experimental import pallas as pl
from jax.experimental.pallas import tpu as pltpu
from jax.experimental.pallas import tpu_sc as plsc

N = 10000
E = 320000
D = 128
NC = 2
NS = 16
NW = NC * NS
EPT = E // NW
CH = 80
NCHUNK = EPT // CH
R = 2000
NB = N // R


@functools.partial(
    pl.kernel,
    out_type=jax.ShapeDtypeStruct((NC, N), jnp.float32),
    mesh=plsc.VectorSubcoreMesh(core_axis_name="c", subcore_axis_name="s"),
    scratch_types=[
        pltpu.VMEM((NCHUNK, CH), jnp.int32),
        pltpu.VMEM((CH,), jnp.float32),
        pltpu.VMEM_SHARED((N,), jnp.float32),
    ],
)
def _deg_kernel(dst_hbm, zeros_hbm, hist_hbm, dstbuf, ones, deg_sh):
    c = lax.axis_index("c")
    s = lax.axis_index("s")
    w = c * NS + s
    pltpu.sync_copy(dst_hbm.at[w], dstbuf)
    for i in range(CH // 16):
        ones[pl.ds(i * 16, 16)] = jnp.ones((16,), jnp.float32)

    @pl.when(s == 0)
    def _():
        pltpu.sync_copy(zeros_hbm, deg_sh)

    plsc.subcore_barrier()

    def chunk(j, carry):
        pltpu.sync_copy(ones, deg_sh.at[dstbuf.at[j]], add=True)
        return carry

    lax.fori_loop(0, NCHUNK, chunk, 0)
    plsc.subcore_barrier()

    @pl.when(s == 0)
    def _():
        pltpu.sync_copy(deg_sh, hist_hbm.at[c])


@functools.partial(
    pl.kernel,
    out_type=jax.ShapeDtypeStruct((NC, N, D), jnp.float32),
    mesh=plsc.VectorSubcoreMesh(core_axis_name="c", subcore_axis_name="s"),
    scratch_types=[
        pltpu.VMEM((EPT,), jnp.int32),
        pltpu.VMEM((EPT,), jnp.int32),
        pltpu.VMEM((CH, D), jnp.float32),
        pltpu.VMEM((CH, D), jnp.float32),
        pltpu.VMEM((CH, D), jnp.float32),
        pltpu.VMEM_SHARED((N, D), jnp.float32),
        pltpu.SemaphoreType.DMA,
        pltpu.SemaphoreType.DMA,
        pltpu.SemaphoreType.DMA,
    ],
)
def _edge_kernel(y_hbm, src_hbm, dst_hbm, zeros_hbm, acc_hbm,
                 srcbuf, dstbuf, rows0, rows1, rows2, acc_sh,
                 semg0, semg1, semg2):
    c = lax.axis_index("c")
    s = lax.axis_index("s")
    w = c * NS + s
    pltpu.sync_copy(src_hbm.at[w], srcbuf)
    pltpu.sync_copy(dst_hbm.at[w], dstbuf)

    @pl.when(s < NS - 1)
    def _():
        pltpu.sync_copy(zeros_hbm.at[pl.ds(s * 624, 624)],
                        acc_sh.at[pl.ds(s * 624, 624)])

    @pl.when(s == NS - 1)
    def _():
        pltpu.sync_copy(zeros_hbm.at[pl.ds(9360, 640)],
                        acc_sh.at[pl.ds(9360, 640)])

    plsc.subcore_barrier()

    def sidx(j):
        return srcbuf.at[pl.ds(j * CH, CH)]

    def didx(j):
        return dstbuf.at[pl.ds(j * CH, CH)]

    bufs = ((rows0, semg0), (rows1, semg1), (rows2, semg2))

    for q in range(3):
        pltpu.async_copy(y_hbm.at[sidx(q)], bufs[q][0], bufs[q][1])

    def chunk_triple(p, carry):
        j3 = p * 3
        for q in range(3):
            j = j3 + q
            rows, sem = bufs[q]
            pltpu.make_async_copy(y_hbm.at[sidx(j)], rows, sem).wait()
            pltpu.sync_copy(rows, acc_sh.at[didx(j)], add=True)

            @pl.when(j < NCHUNK - 3)
            def _():
                pltpu.async_copy(y_hbm.at[sidx(j + 3)], rows, sem)

        return carry

    lax.fori_loop(0, NCHUNK // 3, chunk_triple, 0)
    for q in range(NCHUNK % 3):
        j = (NCHUNK // 3) * 3 + q
        rows, sem = bufs[q]
        pltpu.make_async_copy(y_hbm.at[sidx(j)], rows, sem).wait()
        pltpu.sync_copy(rows, acc_sh.at[didx(j)], add=True)
    plsc.subcore_barrier()

    @pl.when(s < NS - 1)
    def _():
        pltpu.sync_copy(acc_sh.at[pl.ds(s * 624, 624)],
                        acc_hbm.at[c].at[pl.ds(s * 624, 624)])

    @pl.when(s == NS - 1)
    def _():
        pltpu.sync_copy(acc_sh.at[pl.ds(9360, 640)],
                        acc_hbm.at[c].at[pl.ds(9360, 640)])


def _dinv_body(hist_ref, dinv_ref):
    deg = jnp.sum(hist_ref[...], axis=0, keepdims=True)
    dinv_ref[...] = lax.rsqrt(deg + 1.0)


def _tc_dinv(hist):
    return pl.pallas_call(
        _dinv_body,
        in_specs=[pl.BlockSpec((NC, N), lambda: (0, 0))],
        out_specs=pl.BlockSpec((1, N), lambda: (0, 0)),
        out_shape=jax.ShapeDtypeStruct((1, N), jnp.float32),
    )(hist)


def _k1_body(x_ref, w_ref, dinv_ref, y_ref):
    y_ref[...] = dinv_ref[...] * jnp.dot(x_ref[...], w_ref[...],
                                         preferred_element_type=jnp.float32)


def _tc_layer1(x, W1, dinv_col):
    return pl.pallas_call(
        _k1_body,
        grid=(NB,),
        in_specs=[
            pl.BlockSpec((R, D), lambda i: (i, 0)),
            pl.BlockSpec((D, D), lambda i: (0, 0)),
            pl.BlockSpec((R, 1), lambda i: (i, 0)),
        ],
        out_specs=pl.BlockSpec((R, D), lambda i: (i, 0)),
        out_shape=jax.ShapeDtypeStruct((N, D), jnp.float32),
    )(x, W1, dinv_col)


def _k2_body(acca_ref, accb_ref, y_ref, dinv_ref, b1_ref, w_ref, y2_ref):
    dinv = dinv_ref[...]
    h = dinv * (acca_ref[...] + accb_ref[...] + y_ref[...]) + b1_ref[...]
    h = jnp.maximum(h, 0.0)
    y2_ref[...] = dinv * jnp.dot(h, w_ref[...],
                                 preferred_element_type=jnp.float32)


def _tc_layer2(acc1, y1, dinv_col, b1, W2):
    return pl.pallas_call(
        _k2_body,
        grid=(NB,),
        in_specs=[
            pl.BlockSpec((R, D), lambda i: (i, 0)),
            pl.BlockSpec((R, D), lambda i: (NB + i, 0)),
            pl.BlockSpec((R, D), lambda i: (i, 0)),
            pl.BlockSpec((R, 1), lambda i: (i, 0)),
            pl.BlockSpec((1, D), lambda i: (0, 0)),
            pl.BlockSpec((D, D), lambda i: (0, 0)),
        ],
        out_specs=pl.BlockSpec((R, D), lambda i: (i, 0)),
        out_shape=jax.ShapeDtypeStruct((N, D), jnp.float32),
    )(acc1, acc1, y1, dinv_col, b1, W2)


def _k3_body(acca_ref, accb_ref, y_ref, dinv_ref, b2_ref, out_ref):
    out_ref[...] = (dinv_ref[...] *
                    (acca_ref[...] + accb_ref[...] + y_ref[...]) + b2_ref[...])


def _tc_final(acc2, y2, dinv_col, b2):
    return pl.pallas_call(
        _k3_body,
        grid=(NB,),
        in_specs=[
            pl.BlockSpec((R, D), lambda i: (i, 0)),
            pl.BlockSpec((R, D), lambda i: (NB + i, 0)),
            pl.BlockSpec((R, D), lambda i: (i, 0)),
            pl.BlockSpec((R, 1), lambda i: (i, 0)),
            pl.BlockSpec((1, D), lambda i: (0, 0)),
        ],
        out_specs=pl.BlockSpec((R, D), lambda i: (i, 0)),
        out_shape=jax.ShapeDtypeStruct((N, D), jnp.float32),
    )(acc2, acc2, y2, dinv_col, b2)


def kernel(x, edge_index, W1, b1, W2, b2):
    src = edge_index[0].astype(jnp.int32)
    dst = edge_index[1].astype(jnp.int32)
    src3 = src.reshape(NW, EPT)
    dst2 = dst.reshape(NW, EPT)
    dst3 = dst.reshape(NW, NCHUNK, CH)
    zeros1 = jnp.zeros((N,), jnp.float32)
    zeros2 = jnp.zeros((N, D), jnp.float32)
    b1r = b1.reshape(1, D)
    b2r = b2.reshape(1, D)

    hist = _deg_kernel(dst3, zeros1)
    dinv_col = _tc_dinv(hist).reshape(N, 1)

    y1 = _tc_layer1(x, W1, dinv_col)
    acc1 = _edge_kernel(y1, src3, dst2, zeros2).reshape(NC * N, D)
    y2 = _tc_layer2(acc1, y1, dinv_col, b1r, W2)
    acc2 = _edge_kernel(y2, src3, dst2, zeros2).reshape(NC * N, D)
    return _tc_final(acc2, y2, dinv_col, b2r)
